# baseline (speedup 1.0000x reference)
import functools

import jax
import jax.numpy as jnp
import numpy as np
from jax.experimental import pallas as pl

B = 8
N = 2048
R1 = 10.0
R2 = 40.0
MAXN = 32
PANEL_ELEM = 4
MAX_PANEL_LEN = 14
MAX_PATTERN = 23
PANEL_ENC = 20
PAT_ENC = 40
NL = 3


def _mlp_apply(params, x):
    inv = 1.0 / np.sqrt(1.0 + 1e-5)
    for (W, b, g, bt) in params:
        x = x @ W.T + b
        x = jnp.maximum(x, 0.0)
        x = g * (x * inv) + bt
    return x


def _fps_body(px, py, pz, n, m):
    iota_n = jax.lax.broadcasted_iota(jnp.int32, (B, n), 1)
    iota_m = jax.lax.broadcasted_iota(jnp.int32, (B, m), 1)
    lx = px[:, 0:1]
    ly = py[:, 0:1]
    lz = pz[:, 0:1]
    cx = jnp.where(iota_m == 0, lx, 0.0)
    cy = jnp.where(iota_m == 0, ly, 0.0)
    cz = jnp.where(iota_m == 0, lz, 0.0)
    dmin0 = jnp.full((B, n), jnp.inf, jnp.float32)

    def body(i, st):
        dmin, lx, ly, lz, cx, cy, cz = st
        dx = px - lx
        dy = py - ly
        dz = pz - lz
        d = (dx * dx + dy * dy) + dz * dz
        dmin = jnp.minimum(dmin, d)
        mx = jnp.max(dmin, axis=1, keepdims=True)
        cand = jnp.where(dmin == mx, iota_n, n)
        idx = jnp.min(cand, axis=1, keepdims=True)
        sel = iota_n == idx
        lx = jnp.max(jnp.where(sel, px, -jnp.inf), axis=1, keepdims=True)
        ly = jnp.max(jnp.where(sel, py, -jnp.inf), axis=1, keepdims=True)
        lz = jnp.max(jnp.where(sel, pz, -jnp.inf), axis=1, keepdims=True)
        oh = iota_m == i
        cx = jnp.where(oh, lx, cx)
        cy = jnp.where(oh, ly, cy)
        cz = jnp.where(oh, lz, cz)
        return (dmin, lx, ly, lz, cx, cy, cz)

    st = jax.lax.fori_loop(1, m, body, (dmin0, lx, ly, lz, cx, cy, cz))
    return st[4], st[5], st[6]


def _fps_kernel(px_ref, py_ref, pz_ref, c1x_ref, c1y_ref, c1z_ref,
                c2x_ref, c2y_ref, c2z_ref):
    px = px_ref[...]
    py = py_ref[...]
    pz = pz_ref[...]
    n = px.shape[1]
    c1x, c1y, c1z = _fps_body(px, py, pz, n, n // 2)
    c1x_ref[...] = c1x
    c1y_ref[...] = c1y
    c1z_ref[...] = c1z
    c2x, c2y, c2z = _fps_body(c1x, c1y, c1z, n // 2, n // 8)
    c2x_ref[...] = c2x
    c2y_ref[...] = c2y
    c2z_ref[...] = c2z


def _fps_pallas(pos):
    n = pos.shape[1]
    f = jax.ShapeDtypeStruct
    outs = pl.pallas_call(
        _fps_kernel,
        out_shape=(f((B, n // 2), jnp.float32),) * 3 + (f((B, n // 8), jnp.float32),) * 3,
    )(pos[:, :, 0], pos[:, :, 1], pos[:, :, 2])
    return outs[0:3], outs[3:6]


def _sel_kernel(px_ref, py_ref, pz_ref, cxs_ref, cys_ref, czs_ref,
                nbr_ref, d2_ref, *, n, nsub, goffset):
    b = pl.program_id(0)
    px = px_ref[b]
    py = py_ref[b]
    pz = pz_ref[b]
    iota_n = jax.lax.broadcasted_iota(jnp.int32, (8, n), 1)
    iota_k = jax.lax.broadcasted_iota(jnp.int32, (8, MAXN), 1)
    off = (b * n).astype(jnp.int32) if goffset else jnp.int32(0)
    for j in range(nsub):
        cx = cxs_ref[0, j]
        cy = cys_ref[0, j]
        cz = czs_ref[0, j]
        dx = px - cx
        dy = py - cy
        dz = pz - cz
        d2 = (dx * dx + dy * dy) + dz * dz

        def body(k, st):
            d2, idxacc, dacc = st
            mn = jnp.min(d2, axis=1, keepdims=True)
            cand = jnp.where(d2 == mn, iota_n, n)
            idx = jnp.min(cand, axis=1, keepdims=True)
            sel = iota_n == idx
            d2 = jnp.where(sel, jnp.inf, d2)
            oh = iota_k == k
            idxacc = jnp.where(oh, idx + off, idxacc)
            dacc = jnp.where(oh, mn, dacc)
            return (d2, idxacc, dacc)

        _, idxacc, dacc = jax.lax.fori_loop(
            0, MAXN, body,
            (d2, jnp.zeros((8, MAXN), jnp.int32), jnp.zeros((8, MAXN), jnp.float32)),
            unroll=4)
        nbr_ref[0, j * 8:(j + 1) * 8, :] = idxacc
        d2_ref[0, j * 8:(j + 1) * 8, :] = dacc


def _select_knn(pos_planes, cent_planes, n, m, goffset=False):
    CHUNK = 64
    nsub = CHUNK // 8
    grid = (B, m // CHUNK)
    tp = [c.reshape(B, m // 8, 8, 1) for c in cent_planes]
    f = jax.ShapeDtypeStruct
    kern = functools.partial(_sel_kernel, n=n, nsub=nsub, goffset=goffset)
    pos_spec = pl.BlockSpec((B, n), lambda b, c: (0, 0))
    cent_spec = pl.BlockSpec((1, nsub, 8, 1), lambda b, c: (b, c, 0, 0))
    out_spec = pl.BlockSpec((1, CHUNK, MAXN), lambda b, c: (b, c, 0))
    nbr, d2 = pl.pallas_call(
        kern,
        grid=grid,
        in_specs=[pos_spec] * 3 + [cent_spec] * 3,
        out_specs=[out_spec, out_spec],
        out_shape=(f((B, m, MAXN), jnp.int32), f((B, m, MAXN), jnp.float32)),
    )(*pos_planes, *tp)
    return nbr, d2


def _set_abstraction(x, pos, r, mlp_params, cent, nbr, d2sel):
    Bb, Nn, _ = pos.shape
    m = cent.shape[1]
    valid = d2sel <= r * r
    nbr_flat = nbr.reshape(Bb, -1)
    npos = jnp.take_along_axis(pos, nbr_flat[:, :, None], axis=1).reshape(Bb, m, MAXN, 3)
    rel = npos - cent[:, :, None, :]
    if x is None:
        feat = rel
    else:
        nx = jnp.take_along_axis(x, nbr_flat[:, :, None], axis=1).reshape(Bb, m, MAXN, x.shape[-1])
        feat = jnp.concatenate([nx, rel], axis=-1)
    msg = _mlp_apply(mlp_params, feat)
    msg = jnp.where(valid[:, :, :, None], msg, -jnp.inf)
    out = jnp.max(msg, axis=2)
    return out


def _lstm_phase(seq_ref, Wih_ref, Whh_ref, b_ref, h0_ref, c0_ref, T):
    for l in range(NL):
        wii, wif, wig, wio = (Wih_ref[l, j] for j in range(4))
        whi, whf, whg, who = (Whh_ref[l, j] for j in range(4))
        bi, bf, bg, bo = (b_ref[l, j] for j in range(4))

        def step(t, carry, wii=wii, wif=wif, wig=wig, wio=wio,
                 whi=whi, whf=whf, whg=whg, who=who, bi=bi, bf=bf, bg=bg, bo=bo):
            h, c = carry
            xt = seq_ref[t]
            gi = jnp.dot(xt, wii, preferred_element_type=jnp.float32) + \
                jnp.dot(h, whi, preferred_element_type=jnp.float32) + bi
            gf = jnp.dot(xt, wif, preferred_element_type=jnp.float32) + \
                jnp.dot(h, whf, preferred_element_type=jnp.float32) + bf
            gg = jnp.dot(xt, wig, preferred_element_type=jnp.float32) + \
                jnp.dot(h, whg, preferred_element_type=jnp.float32) + bg
            go = jnp.dot(xt, wio, preferred_element_type=jnp.float32) + \
                jnp.dot(h, who, preferred_element_type=jnp.float32) + bo
            c = jax.nn.sigmoid(gf) * c + jax.nn.sigmoid(gi) * jnp.tanh(gg)
            h = jax.nn.sigmoid(go) * jnp.tanh(c)
            seq_ref[t] = h
            return (h, c)

        jax.lax.fori_loop(0, T, step, (h0_ref[l], c0_ref[l]), unroll=1)


def _dec_kernel(enc_ref, pWih_ref, pWhh_ref, pb_ref, ph0_ref, pc0_ref,
                plinW_ref, plinb_ref,
                qWih_ref, qWhh_ref, qb_ref, qh0_ref, qc0_ref,
                qlinW_ref, qlinb_ref,
                out_ref, seq1_ref, seq2_ref):
    T1 = seq1_ref.shape[0]
    for t in range(T1):
        seq1_ref[t] = enc_ref[...]
    _lstm_phase(seq1_ref, pWih_ref, pWhh_ref, pb_ref, ph0_ref, pc0_ref, T1)
    T2 = seq2_ref.shape[0]
    s1 = seq1_ref[...]
    F = s1.reshape(T1 * s1.shape[1], s1.shape[2])
    P = jnp.dot(F, plinW_ref[...], preferred_element_type=jnp.float32) + plinb_ref[...]
    for t in range(T2):
        seq2_ref[t] = P
    _lstm_phase(seq2_ref, qWih_ref, qWhh_ref, qb_ref, qh0_ref, qc0_ref, T2)
    for t in range(T2):
        out_ref[t] = jnp.dot(seq2_ref[t], qlinW_ref[...],
                             preferred_element_type=jnp.float32) + qlinb_ref[...]


def _split_gates(Wih, Whh, bih, bhh, H):
    Wi = Wih.reshape(4, H, -1).transpose(0, 2, 1)
    Wh = Whh.reshape(4, H, -1).transpose(0, 2, 1)
    b = (bih + bhh).reshape(4, 1, H)
    return Wi, Wh, b


def _lstm_decode_pallas(pat_lstm, pat_lin, pan_lstm, pan_lin, enc):
    from jax.experimental.pallas import tpu as pltpu
    Bsz = enc.shape[0]
    B2 = Bsz * MAX_PATTERN
    std1 = float(np.sqrt(2.0 / (Bsz * PAT_ENC)))
    k1, k2 = jax.random.split(jax.random.key(1))
    ph0 = jax.random.normal(k1, (NL, Bsz, PAT_ENC), jnp.float32) * std1
    pc0 = jax.random.normal(k2, (NL, Bsz, PAT_ENC), jnp.float32) * std1
    std2 = float(np.sqrt(2.0 / (B2 * PANEL_ENC)))
    k3, k4 = jax.random.split(jax.random.key(2))
    qh0 = jax.random.normal(k3, (NL, B2, PANEL_ENC), jnp.float32) * std2
    qc0 = jax.random.normal(k4, (NL, B2, PANEL_ENC), jnp.float32) * std2
    perm = (qh0.reshape(NL, Bsz, MAX_PATTERN, PANEL_ENC)
            .transpose(0, 2, 1, 3).reshape(NL, B2, PANEL_ENC))
    permc = (qc0.reshape(NL, Bsz, MAX_PATTERN, PANEL_ENC)
             .transpose(0, 2, 1, 3).reshape(NL, B2, PANEL_ENC))

    pWi, pWh, pb = jax.tree.map(
        lambda *xs: jnp.stack(xs),
        *[_split_gates(*pat_lstm[l], PAT_ENC) for l in range(NL)])
    qWi, qWh, qb = jax.tree.map(
        lambda *xs: jnp.stack(xs),
        *[_split_gates(*pan_lstm[l], PANEL_ENC) for l in range(NL)])

    out = pl.pallas_call(
        _dec_kernel,
        out_shape=jax.ShapeDtypeStruct((MAX_PANEL_LEN, B2, PANEL_ELEM), jnp.float32),
        scratch_shapes=[
            pltpu.VMEM((MAX_PATTERN, Bsz, PAT_ENC), jnp.float32),
            pltpu.VMEM((MAX_PANEL_LEN, B2, PANEL_ENC), jnp.float32),
        ],
    )(enc, pWi, pWh, pb, ph0, pc0,
      pat_lin[0].T, pat_lin[1][None, :],
      qWi, qWh, qb, perm, permc,
      pan_lin[0].T, pan_lin[1][None, :])
    return out.reshape(MAX_PANEL_LEN, MAX_PATTERN, Bsz, PANEL_ELEM).transpose(2, 1, 0, 3)


def _head_kernel(g_ref, w1_ref, b1_ref, w2_ref, b2_ref, w3_ref, b3_ref, out_ref):
    h = jnp.maximum(jnp.dot(g_ref[...], w1_ref[...].T,
                            preferred_element_type=jnp.float32) + b1_ref[...], 0.0)
    h = jnp.maximum(jnp.dot(h, w2_ref[...].T,
                            preferred_element_type=jnp.float32) + b2_ref[...], 0.0)
    out_ref[...] = jnp.dot(h, w3_ref[...].T,
                           preferred_element_type=jnp.float32) + b3_ref[...]


def kernel(positions_batch, mlp1, mlp2, mlp3, lin1, lin2, lin3, pat_lstm, pat_lin, pan_lstm, pan_lin):
    Bb = positions_batch.shape[0]
    c1p, c2p = _fps_pallas(positions_batch)
    cent1 = jnp.stack(c1p, axis=-1)
    cent2 = jnp.stack(c2p, axis=-1)
    pos_planes = [positions_batch[:, :, i] for i in range(3)]
    nbr1, d2s1 = _select_knn(pos_planes, c1p, N, N // 2)
    nbr2, d2s2 = _select_knn(c1p, c2p, N // 2, N // 8)
    x1 = _set_abstraction(None, positions_batch, R1, mlp1, cent1, nbr1, d2s1)
    x2 = _set_abstraction(x1, cent1, R2, mlp2, cent2, nbr2, d2s2)
    pos2 = cent2
    g = jnp.max(_mlp_apply(mlp3, jnp.concatenate([x2, pos2], axis=-1)), axis=1)
    enc = pl.pallas_call(
        _head_kernel,
        out_shape=jax.ShapeDtypeStruct((Bb, PAT_ENC), jnp.float32),
    )(g, lin1[0], lin1[1][None, :], lin2[0], lin2[1][None, :], lin3[0], lin3[1][None, :])
    return _lstm_decode_pallas(pat_lstm, pat_lin, pan_lstm, pan_lin, enc)


# speedup vs baseline: 1.5663x; 1.5663x over previous
import functools

import jax
import jax.numpy as jnp
import numpy as np
from jax import lax
from jax.experimental import pallas as pl
from jax.experimental.pallas import tpu as pltpu
from jax.experimental.pallas import tpu_sc as plsc

B = 8
N = 2048
R1 = 10.0
R2 = 40.0
MAXN = 32
PANEL_ELEM = 4
MAX_PANEL_LEN = 14
MAX_PATTERN = 23
PANEL_ENC = 20
PAT_ENC = 40
NL = 3


def _mlp_apply(params, x):
    inv = 1.0 / np.sqrt(1.0 + 1e-5)
    for (W, b, g, bt) in params:
        x = x @ W.T + b
        x = jnp.maximum(x, 0.0)
        x = g * (x * inv) + bt
    return x


def _fps_body(px, py, pz, n, m):
    iota_n = jax.lax.broadcasted_iota(jnp.int32, (B, n), 1)
    iota_m = jax.lax.broadcasted_iota(jnp.int32, (B, m), 1)
    lx = px[:, 0:1]
    ly = py[:, 0:1]
    lz = pz[:, 0:1]
    cx = jnp.where(iota_m == 0, lx, 0.0)
    cy = jnp.where(iota_m == 0, ly, 0.0)
    cz = jnp.where(iota_m == 0, lz, 0.0)
    dmin0 = jnp.full((B, n), jnp.inf, jnp.float32)

    def body(i, st):
        dmin, lx, ly, lz, cx, cy, cz = st
        dx = px - lx
        dy = py - ly
        dz = pz - lz
        d = (dx * dx + dy * dy) + dz * dz
        dmin = jnp.minimum(dmin, d)
        mx = jnp.max(dmin, axis=1, keepdims=True)
        cand = jnp.where(dmin == mx, iota_n, n)
        idx = jnp.min(cand, axis=1, keepdims=True)
        sel = iota_n == idx
        lx = jnp.max(jnp.where(sel, px, -jnp.inf), axis=1, keepdims=True)
        ly = jnp.max(jnp.where(sel, py, -jnp.inf), axis=1, keepdims=True)
        lz = jnp.max(jnp.where(sel, pz, -jnp.inf), axis=1, keepdims=True)
        oh = iota_m == i
        cx = jnp.where(oh, lx, cx)
        cy = jnp.where(oh, ly, cy)
        cz = jnp.where(oh, lz, cz)
        return (dmin, lx, ly, lz, cx, cy, cz)

    st = jax.lax.fori_loop(1, m, body, (dmin0, lx, ly, lz, cx, cy, cz))
    return st[4], st[5], st[6]


def _fps_kernel(px_ref, py_ref, pz_ref, c1x_ref, c1y_ref, c1z_ref,
                c2x_ref, c2y_ref, c2z_ref):
    px = px_ref[...]
    py = py_ref[...]
    pz = pz_ref[...]
    n = px.shape[1]
    c1x, c1y, c1z = _fps_body(px, py, pz, n, n // 2)
    c1x_ref[...] = c1x
    c1y_ref[...] = c1y
    c1z_ref[...] = c1z
    c2x, c2y, c2z = _fps_body(c1x, c1y, c1z, n // 2, n // 8)
    c2x_ref[...] = c2x
    c2y_ref[...] = c2y
    c2z_ref[...] = c2z


def _fps_pallas(pos):
    n = pos.shape[1]
    f = jax.ShapeDtypeStruct
    outs = pl.pallas_call(
        _fps_kernel,
        out_shape=(f((B, n // 2), jnp.float32),) * 3 + (f((B, n // 8), jnp.float32),) * 3,
    )(pos[:, :, 0], pos[:, :, 1], pos[:, :, 2])
    return outs[0:3], outs[3:6]


def _sel_kernel(px_ref, py_ref, pz_ref, cxs_ref, cys_ref, czs_ref,
                nbr_ref, d2_ref, *, n, nsub, goffset):
    b = pl.program_id(0)
    px = px_ref[b]
    py = py_ref[b]
    pz = pz_ref[b]
    iota_n = jax.lax.broadcasted_iota(jnp.int32, (8, n), 1)
    iota_k = jax.lax.broadcasted_iota(jnp.int32, (8, MAXN), 1)
    off = (b * n).astype(jnp.int32) if goffset else jnp.int32(0)
    for j in range(nsub):
        cx = cxs_ref[0, j]
        cy = cys_ref[0, j]
        cz = czs_ref[0, j]
        dx = px - cx
        dy = py - cy
        dz = pz - cz
        d2 = (dx * dx + dy * dy) + dz * dz

        def body(k, st):
            d2, idxacc, dacc = st
            mn = jnp.min(d2, axis=1, keepdims=True)
            cand = jnp.where(d2 == mn, iota_n, n)
            idx = jnp.min(cand, axis=1, keepdims=True)
            sel = iota_n == idx
            d2 = jnp.where(sel, jnp.inf, d2)
            oh = iota_k == k
            idxacc = jnp.where(oh, idx + off, idxacc)
            dacc = jnp.where(oh, mn, dacc)
            return (d2, idxacc, dacc)

        _, idxacc, dacc = jax.lax.fori_loop(
            0, MAXN, body,
            (d2, jnp.zeros((8, MAXN), jnp.int32), jnp.zeros((8, MAXN), jnp.float32)),
            unroll=4)
        nbr_ref[0, j * 8:(j + 1) * 8, :] = idxacc
        d2_ref[0, j * 8:(j + 1) * 8, :] = dacc


def _select_knn(pos_planes, cent_planes, n, m, goffset=False):
    CHUNK = 64
    nsub = CHUNK // 8
    grid = (B, m // CHUNK)
    tp = [c.reshape(B, m // 8, 8, 1) for c in cent_planes]
    f = jax.ShapeDtypeStruct
    kern = functools.partial(_sel_kernel, n=n, nsub=nsub, goffset=goffset)
    pos_spec = pl.BlockSpec((B, n), lambda b, c: (0, 0))
    cent_spec = pl.BlockSpec((1, nsub, 8, 1), lambda b, c: (b, c, 0, 0))
    out_spec = pl.BlockSpec((1, CHUNK, MAXN), lambda b, c: (b, c, 0))
    nbr, d2 = pl.pallas_call(
        kern,
        grid=grid,
        in_specs=[pos_spec] * 3 + [cent_spec] * 3,
        out_specs=[out_spec, out_spec],
        out_shape=(f((B, m, MAXN), jnp.int32), f((B, m, MAXN), jnp.float32)),
    )(*pos_planes, *tp)
    return nbr, d2


def _sc_gather(table, idx2d, D, kch):
    nrow = idx2d.shape[0]
    NW = 32
    rpw = nrow // NW
    nchunk = rpw // kch
    mesh = plsc.VectorSubcoreMesh(core_axis_name="c", subcore_axis_name="s")

    @functools.partial(
        pl.kernel, mesh=mesh,
        out_type=jax.ShapeDtypeStruct((nrow, 128, D), jnp.float32),
        scratch_types=[
            pltpu.VMEM((kch, 128), jnp.int32),
            pltpu.VMEM((kch, 128, D), jnp.float32),
            pltpu.SemaphoreType.DMA,
        ],
    )
    def k(table_hbm, idx_hbm, out_hbm, idx_v, rows_v, sem):
        wid = lax.axis_index("s") * 2 + lax.axis_index("c")
        base = wid * rpw

        def chunk(i, carry):
            r0 = base + i * kch
            pltpu.sync_copy(idx_hbm.at[pl.ds(r0, kch)], idx_v)
            cps = [pltpu.async_copy(table_hbm.at[idx_v.at[j]], rows_v.at[j], sem)
                   for j in range(kch)]
            for c in cps:
                c.wait()
            pltpu.sync_copy(rows_v, out_hbm.at[pl.ds(r0, kch)])
            return carry

        lax.fori_loop(0, nchunk, chunk, 0)

    return k(table, idx2d)


def _gather_rows(table, idx, D, kch):
    R = idx.shape[0]
    out = _sc_gather(table, idx.reshape(R // 128, 128), D, kch)
    return out.reshape(R, D)


def _set_abstraction(x, pos, r, mlp_params, cent, nbr, d2sel):
    Bb, Nn, _ = pos.shape
    m = cent.shape[1]
    valid = d2sel <= r * r
    nbr_flat = nbr.reshape(-1)
    if x is None:
        table = jnp.pad(pos.reshape(Bb * Nn, 3), ((0, 0), (0, 125)))
        g = _gather_rows(table, nbr_flat, 128, 4).reshape(Bb, m, MAXN, 128)
        feat = g[..., :3] - cent[:, :, None, :]
    else:
        C = x.shape[-1]
        table = jnp.pad(
            jnp.concatenate([x, pos], axis=-1).reshape(Bb * Nn, C + 3),
            ((0, 0), (0, 253 - C)))
        g = _gather_rows(table, nbr_flat, 256, 2).reshape(Bb, m, MAXN, 256)
        rel = g[..., C:C + 3] - cent[:, :, None, :]
        feat = jnp.concatenate([g[..., :C], rel], axis=-1)
    msg = _mlp_apply(mlp_params, feat)
    msg = jnp.where(valid[:, :, :, None], msg, -jnp.inf)
    out = jnp.max(msg, axis=2)
    return out


def _lstm_phase(seq_ref, Wih_ref, Whh_ref, b_ref, h0_ref, c0_ref, T):
    for l in range(NL):
        wii, wif, wig, wio = (Wih_ref[l, j] for j in range(4))
        whi, whf, whg, who = (Whh_ref[l, j] for j in range(4))
        bi, bf, bg, bo = (b_ref[l, j] for j in range(4))

        def step(t, carry, wii=wii, wif=wif, wig=wig, wio=wio,
                 whi=whi, whf=whf, whg=whg, who=who, bi=bi, bf=bf, bg=bg, bo=bo):
            h, c = carry
            xt = seq_ref[t]
            gi = jnp.dot(xt, wii, preferred_element_type=jnp.float32) + \
                jnp.dot(h, whi, preferred_element_type=jnp.float32) + bi
            gf = jnp.dot(xt, wif, preferred_element_type=jnp.float32) + \
                jnp.dot(h, whf, preferred_element_type=jnp.float32) + bf
            gg = jnp.dot(xt, wig, preferred_element_type=jnp.float32) + \
                jnp.dot(h, whg, preferred_element_type=jnp.float32) + bg
            go = jnp.dot(xt, wio, preferred_element_type=jnp.float32) + \
                jnp.dot(h, who, preferred_element_type=jnp.float32) + bo
            c = jax.nn.sigmoid(gf) * c + jax.nn.sigmoid(gi) * jnp.tanh(gg)
            h = jax.nn.sigmoid(go) * jnp.tanh(c)
            seq_ref[t] = h
            return (h, c)

        jax.lax.fori_loop(0, T, step, (h0_ref[l], c0_ref[l]), unroll=1)


def _dec_kernel(enc_ref, pWih_ref, pWhh_ref, pb_ref, ph0_ref, pc0_ref,
                plinW_ref, plinb_ref,
                qWih_ref, qWhh_ref, qb_ref, qh0_ref, qc0_ref,
                qlinW_ref, qlinb_ref,
                out_ref, seq1_ref, seq2_ref):
    T1 = seq1_ref.shape[0]
    for t in range(T1):
        seq1_ref[t] = enc_ref[...]
    _lstm_phase(seq1_ref, pWih_ref, pWhh_ref, pb_ref, ph0_ref, pc0_ref, T1)
    T2 = seq2_ref.shape[0]
    s1 = seq1_ref[...]
    F = s1.reshape(T1 * s1.shape[1], s1.shape[2])
    P = jnp.dot(F, plinW_ref[...], preferred_element_type=jnp.float32) + plinb_ref[...]
    for t in range(T2):
        seq2_ref[t] = P
    _lstm_phase(seq2_ref, qWih_ref, qWhh_ref, qb_ref, qh0_ref, qc0_ref, T2)
    for t in range(T2):
        out_ref[t] = jnp.dot(seq2_ref[t], qlinW_ref[...],
                             preferred_element_type=jnp.float32) + qlinb_ref[...]


def _split_gates(Wih, Whh, bih, bhh, H):
    Wi = Wih.reshape(4, H, -1).transpose(0, 2, 1)
    Wh = Whh.reshape(4, H, -1).transpose(0, 2, 1)
    b = (bih + bhh).reshape(4, 1, H)
    return Wi, Wh, b


def _lstm_decode_pallas(pat_lstm, pat_lin, pan_lstm, pan_lin, enc):
    from jax.experimental.pallas import tpu as pltpu
    Bsz = enc.shape[0]
    B2 = Bsz * MAX_PATTERN
    std1 = float(np.sqrt(2.0 / (Bsz * PAT_ENC)))
    k1, k2 = jax.random.split(jax.random.key(1))
    ph0 = jax.random.normal(k1, (NL, Bsz, PAT_ENC), jnp.float32) * std1
    pc0 = jax.random.normal(k2, (NL, Bsz, PAT_ENC), jnp.float32) * std1
    std2 = float(np.sqrt(2.0 / (B2 * PANEL_ENC)))
    k3, k4 = jax.random.split(jax.random.key(2))
    qh0 = jax.random.normal(k3, (NL, B2, PANEL_ENC), jnp.float32) * std2
    qc0 = jax.random.normal(k4, (NL, B2, PANEL_ENC), jnp.float32) * std2
    perm = (qh0.reshape(NL, Bsz, MAX_PATTERN, PANEL_ENC)
            .transpose(0, 2, 1, 3).reshape(NL, B2, PANEL_ENC))
    permc = (qc0.reshape(NL, Bsz, MAX_PATTERN, PANEL_ENC)
             .transpose(0, 2, 1, 3).reshape(NL, B2, PANEL_ENC))

    pWi, pWh, pb = jax.tree.map(
        lambda *xs: jnp.stack(xs),
        *[_split_gates(*pat_lstm[l], PAT_ENC) for l in range(NL)])
    qWi, qWh, qb = jax.tree.map(
        lambda *xs: jnp.stack(xs),
        *[_split_gates(*pan_lstm[l], PANEL_ENC) for l in range(NL)])

    out = pl.pallas_call(
        _dec_kernel,
        out_shape=jax.ShapeDtypeStruct((MAX_PANEL_LEN, B2, PANEL_ELEM), jnp.float32),
        scratch_shapes=[
            pltpu.VMEM((MAX_PATTERN, Bsz, PAT_ENC), jnp.float32),
            pltpu.VMEM((MAX_PANEL_LEN, B2, PANEL_ENC), jnp.float32),
        ],
    )(enc, pWi, pWh, pb, ph0, pc0,
      pat_lin[0].T, pat_lin[1][None, :],
      qWi, qWh, qb, perm, permc,
      pan_lin[0].T, pan_lin[1][None, :])
    return out.reshape(MAX_PANEL_LEN, MAX_PATTERN, Bsz, PANEL_ELEM).transpose(2, 1, 0, 3)


def _head_kernel(g_ref, w1_ref, b1_ref, w2_ref, b2_ref, w3_ref, b3_ref, out_ref):
    h = jnp.maximum(jnp.dot(g_ref[...], w1_ref[...].T,
                            preferred_element_type=jnp.float32) + b1_ref[...], 0.0)
    h = jnp.maximum(jnp.dot(h, w2_ref[...].T,
                            preferred_element_type=jnp.float32) + b2_ref[...], 0.0)
    out_ref[...] = jnp.dot(h, w3_ref[...].T,
                           preferred_element_type=jnp.float32) + b3_ref[...]


def kernel(positions_batch, mlp1, mlp2, mlp3, lin1, lin2, lin3, pat_lstm, pat_lin, pan_lstm, pan_lin):
    Bb = positions_batch.shape[0]
    c1p, c2p = _fps_pallas(positions_batch)
    cent1 = jnp.stack(c1p, axis=-1)
    cent2 = jnp.stack(c2p, axis=-1)
    pos_planes = [positions_batch[:, :, i] for i in range(3)]
    nbr1, d2s1 = _select_knn(pos_planes, c1p, N, N // 2, goffset=True)
    nbr2, d2s2 = _select_knn(c1p, c2p, N // 2, N // 8, goffset=True)
    x1 = _set_abstraction(None, positions_batch, R1, mlp1, cent1, nbr1, d2s1)
    x2 = _set_abstraction(x1, cent1, R2, mlp2, cent2, nbr2, d2s2)
    pos2 = cent2
    g = jnp.max(_mlp_apply(mlp3, jnp.concatenate([x2, pos2], axis=-1)), axis=1)
    enc = pl.pallas_call(
        _head_kernel,
        out_shape=jax.ShapeDtypeStruct((Bb, PAT_ENC), jnp.float32),
    )(g, lin1[0], lin1[1][None, :], lin2[0], lin2[1][None, :], lin3[0], lin3[1][None, :])
    return _lstm_decode_pallas(pat_lstm, pat_lin, pan_lstm, pan_lin, enc)


# speedup vs baseline: 7.8650x; 5.0215x over previous
import functools

import jax
import jax.numpy as jnp
import numpy as np
from jax import lax
from jax.experimental import pallas as pl
from jax.experimental.pallas import tpu as pltpu
from jax.experimental.pallas import tpu_sc as plsc

B = 8
N = 2048
R1 = 10.0
R2 = 40.0
MAXN = 32
PANEL_ELEM = 4
MAX_PANEL_LEN = 14
MAX_PATTERN = 23
PANEL_ENC = 20
PAT_ENC = 40
NL = 3


def _mlp_apply(params, x):
    inv = 1.0 / np.sqrt(1.0 + 1e-5)
    for (W, b, g, bt) in params:
        x = x @ W.T + b
        x = jnp.maximum(x, 0.0)
        x = g * (x * inv) + bt
    return x


def _fps_body(px, py, pz, n, m):
    iota_n = jax.lax.broadcasted_iota(jnp.int32, (B, n), 1)
    iota_m = jax.lax.broadcasted_iota(jnp.int32, (B, m), 1)
    lx = px[:, 0:1]
    ly = py[:, 0:1]
    lz = pz[:, 0:1]
    cx = jnp.where(iota_m == 0, lx, 0.0)
    cy = jnp.where(iota_m == 0, ly, 0.0)
    cz = jnp.where(iota_m == 0, lz, 0.0)
    dmin0 = jnp.full((B, n), jnp.inf, jnp.float32)

    def body(i, st):
        dmin, lx, ly, lz, cx, cy, cz = st
        dx = px - lx
        dy = py - ly
        dz = pz - lz
        d = (dx * dx + dy * dy) + dz * dz
        dmin = jnp.minimum(dmin, d)
        mx = jnp.max(dmin, axis=1, keepdims=True)
        cand = jnp.where(dmin == mx, iota_n, n)
        idx = jnp.min(cand, axis=1, keepdims=True)
        sel = iota_n == idx
        lx = jnp.max(jnp.where(sel, px, -jnp.inf), axis=1, keepdims=True)
        ly = jnp.max(jnp.where(sel, py, -jnp.inf), axis=1, keepdims=True)
        lz = jnp.max(jnp.where(sel, pz, -jnp.inf), axis=1, keepdims=True)
        oh = iota_m == i
        cx = jnp.where(oh, lx, cx)
        cy = jnp.where(oh, ly, cy)
        cz = jnp.where(oh, lz, cz)
        return (dmin, lx, ly, lz, cx, cy, cz)

    st = jax.lax.fori_loop(1, m, body, (dmin0, lx, ly, lz, cx, cy, cz))
    return st[4], st[5], st[6]


def _fps_kernel(px_ref, py_ref, pz_ref, c1x_ref, c1y_ref, c1z_ref,
                c2x_ref, c2y_ref, c2z_ref):
    px = px_ref[...]
    py = py_ref[...]
    pz = pz_ref[...]
    n = px.shape[1]
    c1x, c1y, c1z = _fps_body(px, py, pz, n, n // 2)
    c1x_ref[...] = c1x
    c1y_ref[...] = c1y
    c1z_ref[...] = c1z
    c2x, c2y, c2z = _fps_body(c1x, c1y, c1z, n // 2, n // 8)
    c2x_ref[...] = c2x
    c2y_ref[...] = c2y
    c2z_ref[...] = c2z


def _fps_pallas(pos):
    n = pos.shape[1]
    f = jax.ShapeDtypeStruct
    outs = pl.pallas_call(
        _fps_kernel,
        out_shape=(f((B, n // 2), jnp.float32),) * 3 + (f((B, n // 8), jnp.float32),) * 3,
    )(pos[:, :, 0], pos[:, :, 1], pos[:, :, 2])
    return outs[0:3], outs[3:6]


def _sel_kernel(px_ref, py_ref, pz_ref, cxs_ref, cys_ref, czs_ref,
                nbr_ref, d2_ref, d2s_ref, *, n, nsub, goffset):
    b = pl.program_id(0)
    px = px_ref[b]
    py = py_ref[b]
    pz = pz_ref[b]
    iota_n = jax.lax.broadcasted_iota(jnp.int32, (8, n), 1)
    iota_k = jax.lax.broadcasted_iota(jnp.int32, (8, MAXN), 1)
    off = (b * n).astype(jnp.int32) if goffset else jnp.int32(0)
    for j in range(nsub):
        cx = cxs_ref[0, j]
        cy = cys_ref[0, j]
        cz = czs_ref[0, j]
        dx = px - cx
        dy = py - cy
        dz = pz - cz
        d2s_ref[j] = (dx * dx + dy * dy) + dz * dz

    zi = jnp.zeros((8, MAXN), jnp.int32)
    zf = jnp.zeros((8, MAXN), jnp.float32)

    def body(k, accs):
        oh = iota_k == k
        new = []
        for j in range(nsub):
            idxacc, dacc = accs[j]
            d2 = d2s_ref[j]
            mn = jnp.min(d2, axis=1, keepdims=True)
            hit = d2 == mn
            cand = jnp.where(hit, iota_n, n)
            idx = jnp.min(cand, axis=1, keepdims=True)
            d2s_ref[j] = jnp.where(hit & (cand == idx), jnp.inf, d2)
            idxacc = jnp.where(oh, idx + off, idxacc)
            dacc = jnp.where(oh, mn, dacc)
            new.append((idxacc, dacc))
        return tuple(new)

    accs = jax.lax.fori_loop(0, MAXN, body, tuple((zi, zf) for _ in range(nsub)))
    for j in range(nsub):
        nbr_ref[0, j * 8:(j + 1) * 8, :] = accs[j][0]
        d2_ref[0, j * 8:(j + 1) * 8, :] = accs[j][1]


def _select_knn(pos_planes, cent_planes, n, m, goffset=False):
    CHUNK = 128
    nsub = CHUNK // 8
    grid = (B, m // CHUNK)
    tp = [c.reshape(B, m // 8, 8, 1) for c in cent_planes]
    f = jax.ShapeDtypeStruct
    kern = functools.partial(_sel_kernel, n=n, nsub=nsub, goffset=goffset)
    pos_spec = pl.BlockSpec((B, n), lambda b, c: (0, 0))
    cent_spec = pl.BlockSpec((1, nsub, 8, 1), lambda b, c: (b, c, 0, 0))
    out_spec = pl.BlockSpec((1, CHUNK, MAXN), lambda b, c: (b, c, 0))
    nbr, d2 = pl.pallas_call(
        kern,
        grid=grid,
        in_specs=[pos_spec] * 3 + [cent_spec] * 3,
        out_specs=[out_spec, out_spec],
        out_shape=(f((B, m, MAXN), jnp.int32), f((B, m, MAXN), jnp.float32)),
        scratch_shapes=[pltpu.VMEM((nsub, 8, n), jnp.float32)],
    )(*pos_planes, *tp)
    return nbr, d2


def _sc_gather(table, idx2d, D, kch):
    nrow = idx2d.shape[0]
    NW = 32
    rpw = nrow // NW
    nchunk = rpw // kch
    mesh = plsc.VectorSubcoreMesh(core_axis_name="c", subcore_axis_name="s")

    @functools.partial(
        pl.kernel, mesh=mesh,
        out_type=jax.ShapeDtypeStruct((nrow, 128, D), jnp.float32),
        scratch_types=[
            pltpu.VMEM((kch, 128), jnp.int32),
            pltpu.VMEM((kch, 128, D), jnp.float32),
            pltpu.SemaphoreType.DMA,
        ],
    )
    def k(table_hbm, idx_hbm, out_hbm, idx_v, rows_v, sem):
        wid = lax.axis_index("s") * 2 + lax.axis_index("c")
        base = wid * rpw

        def chunk(i, carry):
            r0 = base + i * kch
            pltpu.sync_copy(idx_hbm.at[pl.ds(r0, kch)], idx_v)
            cps = [pltpu.async_copy(table_hbm.at[idx_v.at[j]], rows_v.at[j], sem)
                   for j in range(kch)]
            for c in cps:
                c.wait()
            pltpu.sync_copy(rows_v, out_hbm.at[pl.ds(r0, kch)])
            return carry

        lax.fori_loop(0, nchunk, chunk, 0)

    return k(table, idx2d)


def _gather_rows(table, idx, D, kch):
    R = idx.shape[0]
    out = _sc_gather(table, idx.reshape(R // 128, 128), D, kch)
    return out.reshape(R, D)


def _set_abstraction(x, pos, r, mlp_params, cent, nbr, d2sel):
    Bb, Nn, _ = pos.shape
    m = cent.shape[1]
    valid = d2sel <= r * r
    nbr_flat = nbr.reshape(-1)
    if x is None:
        table = jnp.pad(pos.reshape(Bb * Nn, 3), ((0, 0), (0, 125)))
        g = _gather_rows(table, nbr_flat, 128, 4).reshape(Bb, m, MAXN, 128)
        feat = g[..., :3] - cent[:, :, None, :]
    else:
        C = x.shape[-1]
        table = jnp.pad(
            jnp.concatenate([x, pos], axis=-1).reshape(Bb * Nn, C + 3),
            ((0, 0), (0, 253 - C)))
        g = _gather_rows(table, nbr_flat, 256, 2).reshape(Bb, m, MAXN, 256)
        rel = g[..., C:C + 3] - cent[:, :, None, :]
        feat = jnp.concatenate([g[..., :C], rel], axis=-1)
    msg = _mlp_apply(mlp_params, feat)
    msg = jnp.where(valid[:, :, :, None], msg, -jnp.inf)
    out = jnp.max(msg, axis=2)
    return out


def _lstm_phase(seq_ref, Wih_ref, Whh_ref, b_ref, h0_ref, c0_ref, T):
    for l in range(NL):
        wii, wif, wig, wio = (Wih_ref[l, j] for j in range(4))
        whi, whf, whg, who = (Whh_ref[l, j] for j in range(4))
        bi, bf, bg, bo = (b_ref[l, j] for j in range(4))

        def step(t, carry, wii=wii, wif=wif, wig=wig, wio=wio,
                 whi=whi, whf=whf, whg=whg, who=who, bi=bi, bf=bf, bg=bg, bo=bo):
            h, c = carry
            xt = seq_ref[t]
            gi = jnp.dot(xt, wii, preferred_element_type=jnp.float32) + \
                jnp.dot(h, whi, preferred_element_type=jnp.float32) + bi
            gf = jnp.dot(xt, wif, preferred_element_type=jnp.float32) + \
                jnp.dot(h, whf, preferred_element_type=jnp.float32) + bf
            gg = jnp.dot(xt, wig, preferred_element_type=jnp.float32) + \
                jnp.dot(h, whg, preferred_element_type=jnp.float32) + bg
            go = jnp.dot(xt, wio, preferred_element_type=jnp.float32) + \
                jnp.dot(h, who, preferred_element_type=jnp.float32) + bo
            c = jax.nn.sigmoid(gf) * c + jax.nn.sigmoid(gi) * jnp.tanh(gg)
            h = jax.nn.sigmoid(go) * jnp.tanh(c)
            seq_ref[t] = h
            return (h, c)

        jax.lax.fori_loop(0, T, step, (h0_ref[l], c0_ref[l]), unroll=1)


def _dec_kernel(enc_ref, pWih_ref, pWhh_ref, pb_ref, ph0_ref, pc0_ref,
                plinW_ref, plinb_ref,
                qWih_ref, qWhh_ref, qb_ref, qh0_ref, qc0_ref,
                qlinW_ref, qlinb_ref,
                out_ref, seq1_ref, seq2_ref):
    T1 = seq1_ref.shape[0]
    for t in range(T1):
        seq1_ref[t] = enc_ref[...]
    _lstm_phase(seq1_ref, pWih_ref, pWhh_ref, pb_ref, ph0_ref, pc0_ref, T1)
    T2 = seq2_ref.shape[0]
    s1 = seq1_ref[...]
    F = s1.reshape(T1 * s1.shape[1], s1.shape[2])
    P = jnp.dot(F, plinW_ref[...], preferred_element_type=jnp.float32) + plinb_ref[...]
    for t in range(T2):
        seq2_ref[t] = P
    _lstm_phase(seq2_ref, qWih_ref, qWhh_ref, qb_ref, qh0_ref, qc0_ref, T2)
    for t in range(T2):
        out_ref[t] = jnp.dot(seq2_ref[t], qlinW_ref[...],
                             preferred_element_type=jnp.float32) + qlinb_ref[...]


def _split_gates(Wih, Whh, bih, bhh, H):
    Wi = Wih.reshape(4, H, -1).transpose(0, 2, 1)
    Wh = Whh.reshape(4, H, -1).transpose(0, 2, 1)
    b = (bih + bhh).reshape(4, 1, H)
    return Wi, Wh, b


def _lstm_decode_pallas(pat_lstm, pat_lin, pan_lstm, pan_lin, enc):
    from jax.experimental.pallas import tpu as pltpu
    Bsz = enc.shape[0]
    B2 = Bsz * MAX_PATTERN
    std1 = float(np.sqrt(2.0 / (Bsz * PAT_ENC)))
    k1, k2 = jax.random.split(jax.random.key(1))
    ph0 = jax.random.normal(k1, (NL, Bsz, PAT_ENC), jnp.float32) * std1
    pc0 = jax.random.normal(k2, (NL, Bsz, PAT_ENC), jnp.float32) * std1
    std2 = float(np.sqrt(2.0 / (B2 * PANEL_ENC)))
    k3, k4 = jax.random.split(jax.random.key(2))
    qh0 = jax.random.normal(k3, (NL, B2, PANEL_ENC), jnp.float32) * std2
    qc0 = jax.random.normal(k4, (NL, B2, PANEL_ENC), jnp.float32) * std2
    perm = (qh0.reshape(NL, Bsz, MAX_PATTERN, PANEL_ENC)
            .transpose(0, 2, 1, 3).reshape(NL, B2, PANEL_ENC))
    permc = (qc0.reshape(NL, Bsz, MAX_PATTERN, PANEL_ENC)
             .transpose(0, 2, 1, 3).reshape(NL, B2, PANEL_ENC))

    pWi, pWh, pb = jax.tree.map(
        lambda *xs: jnp.stack(xs),
        *[_split_gates(*pat_lstm[l], PAT_ENC) for l in range(NL)])
    qWi, qWh, qb = jax.tree.map(
        lambda *xs: jnp.stack(xs),
        *[_split_gates(*pan_lstm[l], PANEL_ENC) for l in range(NL)])

    out = pl.pallas_call(
        _dec_kernel,
        out_shape=jax.ShapeDtypeStruct((MAX_PANEL_LEN, B2, PANEL_ELEM), jnp.float32),
        scratch_shapes=[
            pltpu.VMEM((MAX_PATTERN, Bsz, PAT_ENC), jnp.float32),
            pltpu.VMEM((MAX_PANEL_LEN, B2, PANEL_ENC), jnp.float32),
        ],
    )(enc, pWi, pWh, pb, ph0, pc0,
      pat_lin[0].T, pat_lin[1][None, :],
      qWi, qWh, qb, perm, permc,
      pan_lin[0].T, pan_lin[1][None, :])
    return out.reshape(MAX_PANEL_LEN, MAX_PATTERN, Bsz, PANEL_ELEM).transpose(2, 1, 0, 3)


def _head_kernel(g_ref, w1_ref, b1_ref, w2_ref, b2_ref, w3_ref, b3_ref, out_ref):
    h = jnp.maximum(jnp.dot(g_ref[...], w1_ref[...].T,
                            preferred_element_type=jnp.float32) + b1_ref[...], 0.0)
    h = jnp.maximum(jnp.dot(h, w2_ref[...].T,
                            preferred_element_type=jnp.float32) + b2_ref[...], 0.0)
    out_ref[...] = jnp.dot(h, w3_ref[...].T,
                           preferred_element_type=jnp.float32) + b3_ref[...]


def kernel(positions_batch, mlp1, mlp2, mlp3, lin1, lin2, lin3, pat_lstm, pat_lin, pan_lstm, pan_lin):
    Bb = positions_batch.shape[0]
    c1p, c2p = _fps_pallas(positions_batch)
    cent1 = jnp.stack(c1p, axis=-1)
    cent2 = jnp.stack(c2p, axis=-1)
    pos_planes = [positions_batch[:, :, i] for i in range(3)]
    nbr1, d2s1 = _select_knn(pos_planes, c1p, N, N // 2, goffset=True)
    nbr2, d2s2 = _select_knn(c1p, c2p, N // 2, N // 8, goffset=True)
    x1 = _set_abstraction(None, positions_batch, R1, mlp1, cent1, nbr1, d2s1)
    x2 = _set_abstraction(x1, cent1, R2, mlp2, cent2, nbr2, d2s2)
    pos2 = cent2
    g = jnp.max(_mlp_apply(mlp3, jnp.concatenate([x2, pos2], axis=-1)), axis=1)
    enc = pl.pallas_call(
        _head_kernel,
        out_shape=jax.ShapeDtypeStruct((Bb, PAT_ENC), jnp.float32),
    )(g, lin1[0], lin1[1][None, :], lin2[0], lin2[1][None, :], lin3[0], lin3[1][None, :])
    return _lstm_decode_pallas(pat_lstm, pat_lin, pan_lstm, pan_lin, enc)
